# baseline (speedup 1.0000x reference)
import functools

import jax
import jax.numpy as jnp
from jax import lax
from jax.experimental import pallas as pl
from jax.experimental.pallas import tpu as pltpu
from jax.experimental.pallas import tpu_sc as plsc

B = 16384
D = 32
NC = 2
NS = 16
NW = NC * NS
LANE = 128
NBUF = 4


def _make_gather(bpw):
    half = bpw // 2

    def _gather_body(uidx, iidx, t_ug, t_ig, t_um, t_im,
                     out_ug, out_ig, out_um, out_im,
                     idx_uv, idx_iv, stage,
                     buf_ug, buf_ig, buf_um, buf_im, sem):
        wid = lax.axis_index("s") * NC + lax.axis_index("c")
        pltpu.sync_copy(uidx.at[wid], idx_uv)
        pltpu.sync_copy(iidx.at[wid], idx_iv)

        tables = (t_ug, t_ig, t_um, t_im)
        bufs = (buf_ug, buf_ig, buf_um, buf_im)
        f_lo = lax.iota(jnp.int32, 16)
        ones = jnp.ones((16,), jnp.int32)

        def issue(iu, ii, slot):
            qu = pl.multiple_of((iu >> 7) << 7, LANE)
            qi = pl.multiple_of((ii >> 7) << 7, LANE)
            for t, q in enumerate((qu, qi, qu, qi)):
                pltpu.async_copy(tables[t].at[:, pl.ds(q, LANE)],
                                 stage.at[slot, :, pl.ds(t * LANE, LANE)],
                                 sem.at[slot])

        def wait_load(iu, ii, slot):
            pltpu.make_async_copy(out_ug.at[wid], stage.at[slot],
                                  sem.at[slot]).wait()
            ru = iu & (LANE - 1)
            ri = ii & (LANE - 1)
            cols = []
            for t, r in enumerate((ru, ri, ru, ri)):
                rv = ones * r + t * LANE
                for h in range(2):
                    fv = f_lo + 16 * h
                    cols.append(plsc.load_gather(stage.at[slot], [fv, rv]))
            return cols

        def scatter_cols(cols, j):
            jv = ones * (j & (half - 1))
            k = 0
            for t in range(4):
                for h in range(2):
                    fv = f_lo + 16 * h
                    plsc.store_scatter(bufs[t], [fv, jv], cols[k])
                    k += 1

        def flush(lo):
            pltpu.sync_copy(buf_ug, out_ug.at[wid, :, pl.ds(lo, half)])
            pltpu.sync_copy(buf_ig, out_ig.at[wid, :, pl.ds(lo, half)])
            pltpu.sync_copy(buf_um, out_um.at[wid, :, pl.ds(lo, half)])
            pltpu.sync_copy(buf_im, out_im.at[wid, :, pl.ds(lo, half)])

        iu_p = idx_uv[0:16]
        ii_p = idx_iv[0:16]
        prev = []
        for l in range(NBUF):
            iu_l = iu_p[l]
            ii_l = ii_p[l]
            issue(iu_l, ii_l, l)
            prev.append((iu_l, ii_l))

        def group(g, carry):
            iu_vec = idx_uv[pl.ds(g * 16, 16)]
            ii_vec = idx_iv[pl.ds(g * 16, 16)]
            hist = [(carry[2 * k], carry[2 * k + 1]) for k in range(NBUF)]
            for l in range(16):
                j = g * 16 + l
                if l == NBUF:
                    @pl.when(g == (bpw // 16) // 2)
                    def _():
                        flush(0)
                iu_l = iu_vec[l]
                ii_l = ii_vec[l]

                @pl.when(j >= NBUF)
                def _(iu_l=iu_l, ii_l=ii_l, j=j, l=l, old=hist[l]):
                    cols = wait_load(old[0], old[1], l % NBUF)
                    issue(iu_l, ii_l, l % NBUF)
                    scatter_cols(cols, j - NBUF)

                hist.append((iu_l, ii_l))
            return sum((tuple(h) for h in hist[16:16 + NBUF]), ())

        carry = lax.fori_loop(0, bpw // 16, group,
                              sum((tuple(p) for p in prev), ()))

        for k in range(NBUF):
            cols = wait_load(carry[2 * k], carry[2 * k + 1], k)
            scatter_cols(cols, bpw - NBUF + k)
        flush(half)

    out_t = jax.ShapeDtypeStruct((NW, D, bpw), jnp.float32)
    return functools.partial(
        pl.kernel,
        mesh=plsc.VectorSubcoreMesh(core_axis_name="c", subcore_axis_name="s"),
        compiler_params=pltpu.CompilerParams(needs_layout_passes=False),
        out_type=[out_t, out_t, out_t, out_t],
        scratch_types=[
            pltpu.VMEM((bpw,), jnp.int32),
            pltpu.VMEM((bpw,), jnp.int32),
            pltpu.VMEM((NBUF, D, 4 * LANE), jnp.float32),
            pltpu.VMEM((D, half), jnp.float32),
            pltpu.VMEM((D, half), jnp.float32),
            pltpu.VMEM((D, half), jnp.float32),
            pltpu.VMEM((D, half), jnp.float32),
            pltpu.SemaphoreType.DMA((NBUF,)),
        ],
    )(_gather_body)


NSPLIT = 2
BPW = B // NW
BPC = BPW // NSPLIT
_gather = _make_gather(BPC)

TC_SUB = 4


def _dense_body(ug, ig, um, im, w1u, w1i, b1, w2, b2, w3, b3, wg, wh, bo, out):
    for b in range(TC_SUB):
        gmf = ug[b] * ig[b]
        h = (jnp.dot(w1u[...], um[b], preferred_element_type=jnp.float32)
             + jnp.dot(w1i[...], im[b], preferred_element_type=jnp.float32)
             + b1[...])
        h = jnp.maximum(h, 0.0)
        h = jnp.maximum(
            jnp.dot(w2[...], h, preferred_element_type=jnp.float32) + b2[...],
            0.0)
        h = jnp.maximum(
            jnp.dot(w3[...], h, preferred_element_type=jnp.float32) + b3[...],
            0.0)
        logit = (jnp.sum(gmf * wg[...], axis=0, keepdims=True)
                 + jnp.sum(h * wh[...], axis=0, keepdims=True)
                 + bo[...])
        out[b] = jax.nn.sigmoid(logit)


def kernel(user_indices, item_indices, user_emb_gmf, item_emb_gmf,
           user_emb_mlp, item_emb_mlp, W1, b1, W2, b2, W3, b3, Wout, bout):
    uidx = user_indices.astype(jnp.int32).reshape(NW, BPW)
    iidx = item_indices.astype(jnp.int32).reshape(NW, BPW)

    tus = (user_emb_gmf.T, item_emb_gmf.T, user_emb_mlp.T, item_emb_mlp.T)

    w1u = W1[:, :D]
    w1i = W1[:, D:]
    wg = Wout[:, :D].T
    wh = Wout[:, D:].T
    b1c = b1.reshape(-1, 1)
    b2c = b2.reshape(-1, 1)
    b3c = b3.reshape(-1, 1)
    boc = bout.reshape(1, 1)

    def blk(shape):
        return pl.BlockSpec((TC_SUB,) + shape[1:],
                            lambda i: (i,) + (0,) * (len(shape) - 1))

    def full(a):
        return pl.BlockSpec(a.shape, lambda i: tuple(0 for _ in a.shape))

    def dense(ug, ig, um, im):
        return pl.pallas_call(
            _dense_body,
            grid=(NW // TC_SUB,),
            in_specs=[blk(ug.shape), blk(ig.shape), blk(um.shape),
                      blk(im.shape),
                      full(w1u), full(w1i), full(b1c), full(W2), full(b2c),
                      full(W3), full(b3c), full(wg), full(wh), full(boc)],
            out_specs=pl.BlockSpec((TC_SUB, 1, BPC), lambda i: (i, 0, 0)),
            out_shape=jax.ShapeDtypeStruct((NW, 1, BPC), jnp.float32),
        )(ug, ig, um, im, w1u, w1i, b1c, W2, b2c, W3, b3c, wg, wh, boc)

    outs = []
    for s in range(NSPLIT):
        lo = s * BPC
        g = _gather(uidx[:, lo:lo + BPC], iidx[:, lo:lo + BPC], *tus)
        outs.append(dense(*g))

    return jnp.concatenate(outs, axis=2).reshape(B)


# speedup vs baseline: 1.0190x; 1.0190x over previous
import functools

import jax
import jax.numpy as jnp
from jax import lax
from jax.experimental import pallas as pl
from jax.experimental.pallas import tpu as pltpu
from jax.experimental.pallas import tpu_sc as plsc

B = 16384
D = 32
NC = 2
NS = 16
NW = NC * NS
BPW = B // NW
LANE = 128
NBUF = 4
HALF = BPW // 2


def _gather_body(uidx, iidx, t_ug, t_ig, t_um, t_im,
                 out_ug, out_ig, out_um, out_im,
                 idx_uv, idx_iv, stage,
                 buf_ug, buf_ig, buf_um, buf_im, sem):
    wid = lax.axis_index("s") * NC + lax.axis_index("c")
    pltpu.sync_copy(uidx.at[wid], idx_uv)
    pltpu.sync_copy(iidx.at[wid], idx_iv)

    tables = (t_ug, t_ig, t_um, t_im)
    bufs = (buf_ug, buf_ig, buf_um, buf_im)
    f_lo = lax.iota(jnp.int32, 16)
    ones = jnp.ones((16,), jnp.int32)

    def issue(iu, ii, slot):
        qu = pl.multiple_of((iu >> 7) << 7, LANE)
        qi = pl.multiple_of((ii >> 7) << 7, LANE)
        for t, q in enumerate((qu, qi, qu, qi)):
            pltpu.async_copy(tables[t].at[:, pl.ds(q, LANE)],
                             stage.at[slot, :, pl.ds(t * LANE, LANE)],
                             sem.at[slot])

    def wait_load(iu, ii, slot):
        pltpu.make_async_copy(out_ug.at[wid], stage.at[slot],
                              sem.at[slot]).wait()
        ru = iu & (LANE - 1)
        ri = ii & (LANE - 1)
        cols = []
        for t, r in enumerate((ru, ri, ru, ri)):
            rv = ones * r + t * LANE
            for half in range(2):
                fv = f_lo + 16 * half
                cols.append(plsc.load_gather(stage.at[slot], [fv, rv]))
        return cols

    def scatter_cols(cols, j):
        jv = ones * (j & (HALF - 1))
        k = 0
        for t in range(4):
            for half in range(2):
                fv = f_lo + 16 * half
                plsc.store_scatter(bufs[t], [fv, jv], cols[k])
                k += 1

    def wait_extract(iu, ii, j, slot):
        scatter_cols(wait_load(iu, ii, slot), j)

    def flush(lo):
        pltpu.sync_copy(buf_ug, out_ug.at[wid, :, pl.ds(lo, HALF)])
        pltpu.sync_copy(buf_ig, out_ig.at[wid, :, pl.ds(lo, HALF)])
        pltpu.sync_copy(buf_um, out_um.at[wid, :, pl.ds(lo, HALF)])
        pltpu.sync_copy(buf_im, out_im.at[wid, :, pl.ds(lo, HALF)])

    iu_p = idx_uv[0:16]
    ii_p = idx_iv[0:16]
    prev = []
    for l in range(NBUF):
        iu_l = iu_p[l]
        ii_l = ii_p[l]
        issue(iu_l, ii_l, l)
        prev.append((iu_l, ii_l))

    def group(g, carry):
        iu_vec = idx_uv[pl.ds(g * 16, 16)]
        ii_vec = idx_iv[pl.ds(g * 16, 16)]
        hist = [(carry[2 * k], carry[2 * k + 1]) for k in range(NBUF)]
        for l in range(16):
            j = g * 16 + l
            if l == NBUF:
                @pl.when(g == (BPW // 16) // 2)
                def _():
                    flush(0)
            iu_l = iu_vec[l]
            ii_l = ii_vec[l]

            @pl.when(j >= NBUF)
            def _(iu_l=iu_l, ii_l=ii_l, j=j, l=l, old=hist[l]):
                cols = wait_load(old[0], old[1], l % NBUF)
                issue(iu_l, ii_l, l % NBUF)
                scatter_cols(cols, j - NBUF)

            hist.append((iu_l, ii_l))
        return sum((tuple(h) for h in hist[16:16 + NBUF]), ())

    carry = lax.fori_loop(0, BPW // 16, group,
                          sum((tuple(p) for p in prev), ()))

    for k in range(NBUF):
        wait_extract(carry[2 * k], carry[2 * k + 1], BPW - NBUF + k, k)
    flush(HALF)


_out_t = jax.ShapeDtypeStruct((NW, D, BPW), jnp.float32)

_gather = functools.partial(
    pl.kernel,
    mesh=plsc.VectorSubcoreMesh(core_axis_name="c", subcore_axis_name="s"),
    compiler_params=pltpu.CompilerParams(needs_layout_passes=False),
    out_type=[_out_t, _out_t, _out_t, _out_t],
    scratch_types=[
        pltpu.VMEM((BPW,), jnp.int32),
        pltpu.VMEM((BPW,), jnp.int32),
        pltpu.VMEM((NBUF, D, 4 * LANE), jnp.float32),
        pltpu.VMEM((D, HALF), jnp.float32),
        pltpu.VMEM((D, HALF), jnp.float32),
        pltpu.VMEM((D, HALF), jnp.float32),
        pltpu.VMEM((D, HALF), jnp.float32),
        pltpu.SemaphoreType.DMA((NBUF,)),
    ],
)(_gather_body)


TC_SUB = 4


def _dense_body(ug, ig, um, im, w1u, w1i, b1, w2, b2, w3, b3, wg, wh, bo, out):
    for b in range(TC_SUB):
        gmf = ug[b] * ig[b]
        h = (jnp.dot(w1u[...], um[b], preferred_element_type=jnp.float32)
             + jnp.dot(w1i[...], im[b], preferred_element_type=jnp.float32)
             + b1[...])
        h = jnp.maximum(h, 0.0)
        h = jnp.maximum(
            jnp.dot(w2[...], h, preferred_element_type=jnp.float32) + b2[...],
            0.0)
        h = jnp.maximum(
            jnp.dot(w3[...], h, preferred_element_type=jnp.float32) + b3[...],
            0.0)
        logit = (jnp.sum(gmf * wg[...], axis=0, keepdims=True)
                 + jnp.sum(h * wh[...], axis=0, keepdims=True)
                 + bo[...])
        out[b] = jax.nn.sigmoid(logit)


def kernel(user_indices, item_indices, user_emb_gmf, item_emb_gmf,
           user_emb_mlp, item_emb_mlp, W1, b1, W2, b2, W3, b3, Wout, bout):
    uidx = user_indices.astype(jnp.int32).reshape(NW, BPW)
    iidx = item_indices.astype(jnp.int32).reshape(NW, BPW)

    ug, ig, um, im = _gather(uidx, iidx, user_emb_gmf.T, item_emb_gmf.T,
                             user_emb_mlp.T, item_emb_mlp.T)

    w1u = W1[:, :D]
    w1i = W1[:, D:]
    wg = Wout[:, :D].T
    wh = Wout[:, D:].T
    b1c = b1.reshape(-1, 1)
    b2c = b2.reshape(-1, 1)
    b3c = b3.reshape(-1, 1)
    boc = bout.reshape(1, 1)

    def blk(shape):
        return pl.BlockSpec((TC_SUB,) + shape[1:],
                            lambda i: (i,) + (0,) * (len(shape) - 1))

    def full(a):
        return pl.BlockSpec(a.shape, lambda i: tuple(0 for _ in a.shape))

    out = pl.pallas_call(
        _dense_body,
        grid=(NW // TC_SUB,),
        in_specs=[blk(ug.shape), blk(ig.shape), blk(um.shape), blk(im.shape),
                  full(w1u), full(w1i), full(b1c), full(W2), full(b2c),
                  full(W3), full(b3c), full(wg), full(wh), full(boc)],
        out_specs=pl.BlockSpec((TC_SUB, 1, BPW), lambda i: (i, 0, 0)),
        out_shape=jax.ShapeDtypeStruct((NW, 1, BPW), jnp.float32),
    )(ug, ig, um, im, w1u, w1i, b1c, W2, b2c, W3, b3c, wg, wh, boc)

    return out.reshape(B)
